# baseline (speedup 1.0000x reference)
import jax
import jax.numpy as jnp
from jax.experimental import pallas as pl
from jax.experimental.pallas import tpu as pltpu


def _prep_kernel(x_ref, a_ref, ex_ref, xm_ref, ea_ref, am_ref):
    xv = x_ref[...]
    m = jnp.max(xv, axis=1, keepdims=True)
    xm_ref[...] = m
    ex_ref[...] = jnp.exp(xv - m).astype(jnp.bfloat16)
    av = a_ref[...]
    c = jnp.max(av, axis=0, keepdims=True)
    am_ref[...] = c
    ea_ref[...] = jnp.exp(av - c).astype(jnp.bfloat16)


def _mm_kernel(ex_ref, ea_ref, xm_ref, am_ref, o_ref):
    s = jnp.dot(ex_ref[...], ea_ref[...], preferred_element_type=jnp.float32)
    o_ref[...] = jnp.log(s) + xm_ref[...] + am_ref[...]


def kernel(x, A):
    N, D = x.shape
    _, E = A.shape
    f32 = jnp.float32

    nblk = 8
    bn1, be1 = N // nblk, E // nblk
    ex, xm, ea, am = pl.pallas_call(
        _prep_kernel,
        grid=(nblk,),
        in_specs=[
            pl.BlockSpec((bn1, D), lambda i: (i, 0)),
            pl.BlockSpec((D, be1), lambda i: (0, i)),
        ],
        out_specs=[
            pl.BlockSpec((bn1, D), lambda i: (i, 0)),
            pl.BlockSpec((bn1, 1), lambda i: (i, 0)),
            pl.BlockSpec((D, be1), lambda i: (0, i)),
            pl.BlockSpec((1, be1), lambda i: (0, i)),
        ],
        out_shape=[
            jax.ShapeDtypeStruct((N, D), jnp.bfloat16),
            jax.ShapeDtypeStruct((N, 1), f32),
            jax.ShapeDtypeStruct((D, E), jnp.bfloat16),
            jax.ShapeDtypeStruct((1, E), f32),
        ],
        compiler_params=pltpu.CompilerParams(
            dimension_semantics=("parallel",),
        ),
        name="logmmexp_prep",
    )(x, A)

    bn2 = N // 8
    out = pl.pallas_call(
        _mm_kernel,
        grid=(N // bn2,),
        in_specs=[
            pl.BlockSpec((bn2, D), lambda i: (i, 0)),
            pl.BlockSpec((D, E), lambda i: (0, 0)),
            pl.BlockSpec((bn2, 1), lambda i: (i, 0)),
            pl.BlockSpec((1, E), lambda i: (0, 0)),
        ],
        out_specs=pl.BlockSpec((bn2, E), lambda i: (i, 0)),
        out_shape=jax.ShapeDtypeStruct((N, E), f32),
        compiler_params=pltpu.CompilerParams(
            dimension_semantics=("parallel",),
            vmem_limit_bytes=52 * 1024 * 1024,
        ),
        name="logmmexp_mm",
    )(ex, ea, xm, am)
    return out


# speedup vs baseline: 1.1509x; 1.1509x over previous
import jax
import jax.numpy as jnp
from jax.experimental import pallas as pl
from jax.experimental.pallas import tpu as pltpu


def _fused_kernel(x_ref, a_ref, o_ref, ex_ref, xm_ref, ea_ref, am_ref):
    j = pl.program_id(0)
    i = pl.program_id(1)
    bn = x_ref.shape[0]
    rows = pl.ds(i * bn, bn)

    @pl.when(j == 0)
    def _():
        xv = x_ref[...]
        m = jnp.max(xv, axis=1, keepdims=True)
        xm_ref[rows, :] = m
        ex_ref[rows, :] = jnp.exp(xv - m).astype(jnp.bfloat16)

    @pl.when(i == 0)
    def _():
        av = a_ref[...]
        c = jnp.max(av, axis=0, keepdims=True)
        am_ref[...] = c
        ea_ref[...] = jnp.exp(av - c).astype(jnp.bfloat16)

    s = jnp.dot(ex_ref[rows, :], ea_ref[...],
                preferred_element_type=jnp.float32)
    o_ref[...] = jnp.log(s) + xm_ref[rows, :] + am_ref[...]


def kernel(x, A):
    N, D = x.shape
    _, E = A.shape
    bn, be = 1024, 1024
    ni, nj = N // bn, E // be

    return pl.pallas_call(
        _fused_kernel,
        grid=(nj, ni),
        in_specs=[
            pl.BlockSpec((bn, D), lambda j, i: (jnp.where(j == 0, i, ni - 1), 0)),
            pl.BlockSpec((D, be), lambda j, i: (0, j)),
        ],
        out_specs=pl.BlockSpec((bn, be), lambda j, i: (i, j)),
        out_shape=jax.ShapeDtypeStruct((N, E), jnp.float32),
        scratch_shapes=[
            pltpu.VMEM((N, D), jnp.bfloat16),
            pltpu.VMEM((N, 1), jnp.float32),
            pltpu.VMEM((D, be), jnp.bfloat16),
            pltpu.VMEM((1, be), jnp.float32),
        ],
        compiler_params=pltpu.CompilerParams(
            dimension_semantics=("arbitrary", "arbitrary"),
            vmem_limit_bytes=52 * 1024 * 1024,
        ),
        name="logmmexp_fused",
    )(x, A)
